# Initial kernel scaffold; baseline (speedup 1.0000x reference)
import jax
import jax.numpy as jnp
from jax.experimental import pallas as pl


def kernel(inputs, embedding):
    raise NotImplementedError("write your pallas kernel here")


# speedup vs baseline: 1.4662x; 1.4662x over previous
import functools

import jax
import jax.numpy as jnp
from jax import lax
from jax.experimental import pallas as pl
from jax.experimental.pallas import tpu as pltpu
from jax.experimental.pallas import tpu_sc as plsc

_NUM_EMB = 8192
_DIM = 32
_TOKENS = 32 * 1024
_TB = 256


def _nearest_body(x_ref, e_ref, x2_ref, e2_ref, idx_ref, loss_ref):
    pid = pl.program_id(0)
    x = x_ref[...]
    e = e_ref[...]
    xe = lax.dot_general(
        x.astype(jnp.bfloat16),
        e,
        (((1,), (1,)), ((), ())),
        preferred_element_type=jnp.float32,
    )
    scores = (x2_ref[...] + e2_ref[...]) - 2.0 * xe
    cols = lax.broadcasted_iota(jnp.int32, scores.shape, 1)
    half = _NUM_EMB // 2
    s0, s1 = scores[:, :half], scores[:, half:]
    c0, c1 = cols[:, :half], cols[:, half:]
    m0 = jnp.min(s0, axis=1, keepdims=True)
    m1 = jnp.min(s1, axis=1, keepdims=True)
    i0 = jnp.min(jnp.where(s0 == m0, c0, jnp.int32(_NUM_EMB)), axis=1)
    i1 = jnp.min(jnp.where(s1 == m1, c1, jnp.int32(_NUM_EMB)), axis=1)
    acc0 = m0.astype(jnp.bfloat16).astype(jnp.float32)
    take1 = (m1 < acc0)[:, 0]
    idx = jnp.where(take1, i1, i0)
    idx_ref[...] = idx[:, None]

    part = jnp.sum(jnp.where(take1[:, None], m1, m0))

    @pl.when(pid == 0)
    def _():
        loss_ref[0, 0] = 0.0

    loss_ref[0, 0] += part


_nearest = pl.pallas_call(
    _nearest_body,
    grid=(_TOKENS // _TB,),
    in_specs=[
        pl.BlockSpec((_TB, _DIM), lambda i: (i, 0)),
        pl.BlockSpec((_NUM_EMB, _DIM), lambda i: (0, 0)),
        pl.BlockSpec((_TB, 1), lambda i: (i, 0)),
        pl.BlockSpec((1, _NUM_EMB), lambda i: (0, 0)),
    ],
    out_specs=[
        pl.BlockSpec((_TB, 1), lambda i: (i, 0)),
        pl.BlockSpec((1, 1), lambda i: (0, 0), memory_space=pltpu.SMEM),
    ],
    out_shape=[
        jax.ShapeDtypeStruct((_TOKENS, 1), jnp.int32),
        jax.ShapeDtypeStruct((1, 1), jnp.float32),
    ],
)


@functools.lru_cache(maxsize=None)
def _make_sc_gather():
    info = plsc.get_sparse_core_info()
    nc, ns = info.num_cores, info.num_subcores
    nw = nc * ns
    b_per_w = _TOKENS // nw
    mesh = plsc.VectorSubcoreMesh(core_axis_name="c", subcore_axis_name="s")

    @functools.partial(
        pl.kernel,
        mesh=mesh,
        out_type=jax.ShapeDtypeStruct((_TOKENS, _DIM), jnp.float32),
        scratch_types=[
            pltpu.VMEM((b_per_w,), jnp.int32),
            pltpu.VMEM((b_per_w, _DIM), jnp.float32),
            pltpu.SemaphoreType.DMA,
        ],
        compiler_params=pltpu.CompilerParams(use_tc_tiling_on_sc=False),
    )
    def gather_rows(table_hbm, idx_hbm, out_hbm, idx_v, rows_v, sem):
        wid = lax.axis_index("s") * nc + lax.axis_index("c")
        base = wid * b_per_w
        pltpu.sync_copy(idx_hbm.at[pl.ds(base, b_per_w)], idx_v)
        pltpu.async_copy(table_hbm.at[idx_v], rows_v, sem).wait()
        pltpu.sync_copy(rows_v, out_hbm.at[pl.ds(base, b_per_w)])

    return gather_rows


def kernel(inputs, embedding):
    input_shape = inputs.shape
    flat = inputs.reshape(-1, _DIM)
    x2 = jnp.sum(flat**2, axis=1, keepdims=True)
    e2 = jnp.sum(embedding**2, axis=1)[None, :]
    idx2d, loss_acc = _nearest(flat, embedding, x2, e2)
    idx = idx2d.reshape(-1)
    quantized = _make_sc_gather()(embedding, idx)
    loss = (2.0 / flat.size) * loss_acc[0, 0]
    return (
        quantized.reshape(input_shape),
        loss,
        idx2d.reshape(input_shape[0], input_shape[1]),
    )


# speedup vs baseline: 1.5328x; 1.0454x over previous
import functools

import jax
import jax.numpy as jnp
from jax import lax
from jax.experimental import pallas as pl
from jax.experimental.pallas import tpu as pltpu
from jax.experimental.pallas import tpu_sc as plsc

_NUM_EMB = 8192
_DIM = 32
_TOKENS = 32 * 1024
_TB = 512


def _nearest_body(x_ref, e_ref, x2_ref, e2_ref, idx_ref, loss_ref):
    pid = pl.program_id(0)
    x = x_ref[...]
    e = e_ref[...]
    xe = lax.dot_general(
        x.astype(jnp.bfloat16),
        e,
        (((1,), (1,)), ((), ())),
        preferred_element_type=jnp.float32,
    )
    scores = (x2_ref[...] + e2_ref[...]) - 2.0 * xe
    cols = lax.broadcasted_iota(jnp.int32, scores.shape, 1)
    half = _NUM_EMB // 2
    s0, s1 = scores[:, :half], scores[:, half:]
    c0, c1 = cols[:, :half], cols[:, half:]
    m0 = jnp.min(s0, axis=1, keepdims=True)
    m1 = jnp.min(s1, axis=1, keepdims=True)
    i0 = jnp.min(jnp.where(s0 == m0, c0, jnp.int32(_NUM_EMB)), axis=1)
    i1 = jnp.min(jnp.where(s1 == m1, c1, jnp.int32(_NUM_EMB)), axis=1)
    acc0 = m0.astype(jnp.bfloat16).astype(jnp.float32)
    take1 = (m1 < acc0)[:, 0]
    idx = jnp.where(take1, i1, i0)
    idx_ref[...] = idx[:, None]

    part = jnp.sum(jnp.where(take1[:, None], m1, m0))

    @pl.when(pid == 0)
    def _():
        loss_ref[0, 0] = 0.0

    loss_ref[0, 0] += part


_nearest = pl.pallas_call(
    _nearest_body,
    grid=(_TOKENS // _TB,),
    in_specs=[
        pl.BlockSpec((_TB, _DIM), lambda i: (i, 0)),
        pl.BlockSpec((_NUM_EMB, _DIM), lambda i: (0, 0)),
        pl.BlockSpec((_TB, 1), lambda i: (i, 0)),
        pl.BlockSpec((1, _NUM_EMB), lambda i: (0, 0)),
    ],
    out_specs=[
        pl.BlockSpec((_TB, 1), lambda i: (i, 0)),
        pl.BlockSpec((1, 1), lambda i: (0, 0), memory_space=pltpu.SMEM),
    ],
    out_shape=[
        jax.ShapeDtypeStruct((_TOKENS, 1), jnp.int32),
        jax.ShapeDtypeStruct((1, 1), jnp.float32),
    ],
)


@functools.lru_cache(maxsize=None)
def _make_sc_gather():
    info = plsc.get_sparse_core_info()
    nc, ns = info.num_cores, info.num_subcores
    nw = nc * ns
    b_per_w = _TOKENS // nw
    mesh = plsc.VectorSubcoreMesh(core_axis_name="c", subcore_axis_name="s")

    @functools.partial(
        pl.kernel,
        mesh=mesh,
        out_type=jax.ShapeDtypeStruct((_TOKENS, _DIM), jnp.float32),
        scratch_types=[
            pltpu.VMEM((b_per_w,), jnp.int32),
            pltpu.VMEM((b_per_w, _DIM), jnp.float32),
            pltpu.SemaphoreType.DMA,
        ],
        compiler_params=pltpu.CompilerParams(use_tc_tiling_on_sc=False),
    )
    def gather_rows(table_hbm, idx_hbm, out_hbm, idx_v, rows_v, sem):
        wid = lax.axis_index("s") * nc + lax.axis_index("c")
        base = wid * b_per_w
        pltpu.sync_copy(idx_hbm.at[pl.ds(base, b_per_w)], idx_v)
        pltpu.async_copy(table_hbm.at[idx_v], rows_v, sem).wait()
        pltpu.sync_copy(rows_v, out_hbm.at[pl.ds(base, b_per_w)])

    return gather_rows


def kernel(inputs, embedding):
    input_shape = inputs.shape
    flat = inputs.reshape(-1, _DIM)
    x2 = jnp.sum(flat**2, axis=1, keepdims=True)
    e2 = jnp.sum(embedding**2, axis=1)[None, :]
    idx2d, loss_acc = _nearest(flat, embedding, x2, e2)
    idx = idx2d.reshape(-1)
    quantized = _make_sc_gather()(embedding, idx)
    loss = (2.0 / flat.size) * loss_acc[0, 0]
    return (
        quantized.reshape(input_shape),
        loss,
        idx2d.reshape(input_shape[0], input_shape[1]),
    )


# speedup vs baseline: 1.5400x; 1.0047x over previous
import functools

import jax
import jax.numpy as jnp
from jax import lax
from jax.experimental import pallas as pl
from jax.experimental.pallas import tpu as pltpu
from jax.experimental.pallas import tpu_sc as plsc

_NUM_EMB = 8192
_DIM = 32
_TOKENS = 32 * 1024
_TB = 1024


def _nearest_body(x_ref, e_ref, x2_ref, e2_ref, idx_ref, loss_ref):
    pid = pl.program_id(0)
    x = x_ref[...]
    e = e_ref[...]
    xe = lax.dot_general(
        x.astype(jnp.bfloat16),
        e,
        (((1,), (1,)), ((), ())),
        preferred_element_type=jnp.float32,
    )
    scores = (x2_ref[...] + e2_ref[...]) - 2.0 * xe
    cols = lax.broadcasted_iota(jnp.int32, scores.shape, 1)
    half = _NUM_EMB // 2
    s0, s1 = scores[:, :half], scores[:, half:]
    c0, c1 = cols[:, :half], cols[:, half:]
    m0 = jnp.min(s0, axis=1, keepdims=True)
    m1 = jnp.min(s1, axis=1, keepdims=True)
    i0 = jnp.min(jnp.where(s0 == m0, c0, jnp.int32(_NUM_EMB)), axis=1)
    i1 = jnp.min(jnp.where(s1 == m1, c1, jnp.int32(_NUM_EMB)), axis=1)
    acc0 = m0.astype(jnp.bfloat16).astype(jnp.float32)
    take1 = (m1 < acc0)[:, 0]
    idx = jnp.where(take1, i1, i0)
    idx_ref[...] = idx[:, None]

    part = jnp.sum(jnp.where(take1[:, None], m1, m0))

    @pl.when(pid == 0)
    def _():
        loss_ref[0, 0] = 0.0

    loss_ref[0, 0] += part


_nearest = pl.pallas_call(
    _nearest_body,
    grid=(_TOKENS // _TB,),
    in_specs=[
        pl.BlockSpec((_TB, _DIM), lambda i: (i, 0)),
        pl.BlockSpec((_NUM_EMB, _DIM), lambda i: (0, 0)),
        pl.BlockSpec((_TB, 1), lambda i: (i, 0)),
        pl.BlockSpec((1, _NUM_EMB), lambda i: (0, 0)),
    ],
    out_specs=[
        pl.BlockSpec((_TB, 1), lambda i: (i, 0)),
        pl.BlockSpec((1, 1), lambda i: (0, 0), memory_space=pltpu.SMEM),
    ],
    out_shape=[
        jax.ShapeDtypeStruct((_TOKENS, 1), jnp.int32),
        jax.ShapeDtypeStruct((1, 1), jnp.float32),
    ],
    compiler_params=pltpu.CompilerParams(vmem_limit_bytes=100 * 1024 * 1024),
)


@functools.lru_cache(maxsize=None)
def _make_sc_gather():
    info = plsc.get_sparse_core_info()
    nc, ns = info.num_cores, info.num_subcores
    nw = nc * ns
    b_per_w = _TOKENS // nw
    mesh = plsc.VectorSubcoreMesh(core_axis_name="c", subcore_axis_name="s")

    @functools.partial(
        pl.kernel,
        mesh=mesh,
        out_type=jax.ShapeDtypeStruct((_TOKENS, _DIM), jnp.float32),
        scratch_types=[
            pltpu.VMEM((b_per_w,), jnp.int32),
            pltpu.VMEM((b_per_w, _DIM), jnp.float32),
            pltpu.SemaphoreType.DMA,
        ],
        compiler_params=pltpu.CompilerParams(use_tc_tiling_on_sc=False),
    )
    def gather_rows(table_hbm, idx_hbm, out_hbm, idx_v, rows_v, sem):
        wid = lax.axis_index("s") * nc + lax.axis_index("c")
        base = wid * b_per_w
        pltpu.sync_copy(idx_hbm.at[pl.ds(base, b_per_w)], idx_v)
        pltpu.async_copy(table_hbm.at[idx_v], rows_v, sem).wait()
        pltpu.sync_copy(rows_v, out_hbm.at[pl.ds(base, b_per_w)])

    return gather_rows


def kernel(inputs, embedding):
    input_shape = inputs.shape
    flat = inputs.reshape(-1, _DIM)
    x2 = jnp.sum(flat**2, axis=1, keepdims=True)
    e2 = jnp.sum(embedding**2, axis=1)[None, :]
    idx2d, loss_acc = _nearest(flat, embedding, x2, e2)
    idx = idx2d.reshape(-1)
    quantized = _make_sc_gather()(embedding, idx)
    loss = (2.0 / flat.size) * loss_acc[0, 0]
    return (
        quantized.reshape(input_shape),
        loss,
        idx2d.reshape(input_shape[0], input_shape[1]),
    )
